# baseline (speedup 1.0000x reference)
import jax
import jax.numpy as jnp
from jax import lax
from jax.experimental import pallas as pl
from jax.experimental.pallas import tpu as pltpu
from jax.experimental.pallas import tpu_sc as plsc

D = 128
DH = D // 2
NC, NS = 2, 16
NPAD = 10240
ZSTR = NPAD // NS
CH = 128
NB = 4

_mesh = plsc.VectorSubcoreMesh(
    core_axis_name="c", subcore_axis_name="s", num_cores=NC, num_subcores=NS
)
_params = pltpu.CompilerParams(
    needs_layout_passes=False, use_tc_tiling_on_sc=False
)


def _worker():
    return lax.axis_index("c"), lax.axis_index("s")


def _fill_1d(ref, n, val):
    v16 = jnp.full((16,), val, jnp.float32)

    @pl.loop(0, n // 16)
    def _(i):
        ref[pl.ds(i * 16, 16)] = v16


def _make_scatter_kernel(nchunk, gcol, with_counts):
    outs = [
        jax.ShapeDtypeStruct((NPAD, DH), jnp.float32),
        jax.ShapeDtypeStruct((NPAD, DH), jnp.float32),
    ]
    if with_counts:
        outs += [
            jax.ShapeDtypeStruct((NPAD,), jnp.float32),
            jax.ShapeDtypeStruct((NPAD,), jnp.float32),
        ]
    scratch = (
        [pltpu.VMEM_SHARED((NPAD, DH), jnp.float32)]
        + [pltpu.VMEM((NB, 2, CH), jnp.int32)]
        + [pltpu.VMEM((CH, DH), jnp.float32) for _ in range(NB)]
        + [pltpu.SemaphoreType.DMA]
        + [pltpu.SemaphoreType.DMA for _ in range(NB)]
        + [pltpu.SemaphoreType.DMA for _ in range(NB)]
    )
    if with_counts:
        scratch += [
            pltpu.VMEM_SHARED((NPAD,), jnp.float32),
            pltpu.VMEM((CH,), jnp.float32),
            pltpu.VMEM((ZSTR,), jnp.float32),
        ]

    def body(src0_hbm, src1_hbm, pidx_hbm, *rest):
        if with_counts:
            (half0, half1, dcnt_out, bcnt_out, acc, pix, *rest2) = rest
            rows = rest2[:NB]
            isem = rest2[NB]
            gsems = rest2[NB + 1:2 * NB + 1]
            ssems = rest2[2 * NB + 1:3 * NB + 1]
            cnt, ones, zcnt = rest2[3 * NB + 1:]
        else:
            (half0, half1, acc, pix, *rest2) = rest
            rows = rest2[:NB]
            isem = rest2[NB]
            gsems = rest2[NB + 1:2 * NB + 1]
            ssems = rest2[2 * NB + 1:3 * NB + 1]
        c, s = _worker()
        z16 = jnp.zeros((16,), jnp.float32)

        @pl.loop(0, CH * (DH // 16))
        def _(t):
            rows[0][t // (DH // 16), pl.ds((t % (DH // 16)) * 16, 16)] = z16

        @pl.loop(0, ZSTR // CH)
        def _(q):
            pltpu.sync_copy(rows[0], acc.at[pl.ds(s * ZSTR + q * CH, CH)])

        if with_counts:
            _fill_1d(ones, CH, 1.0)
            _fill_1d(zcnt, ZSTR, 0.0)
            pltpu.sync_copy(zcnt, cnt.at[pl.ds(s * ZSTR, ZSTR)])

        plsc.subcore_barrier()

        @pl.loop(0, nchunk // NB)
        def _(gq):
            j0 = gq * NB
            pltpu.async_copy(pidx_hbm.at[s, pl.ds(j0, NB)], pix, isem).wait()
            for b in range(NB):

                @pl.when(c == 0)
                def _(b=b):
                    pltpu.async_copy(src0_hbm.at[pix.at[b, gcol]], rows[b],
                                     gsems[b])

                @pl.when(c == 1)
                def _(b=b):
                    pltpu.async_copy(src1_hbm.at[pix.at[b, gcol]], rows[b],
                                     gsems[b])
            dss = []
            for b in range(NB):
                pltpu.make_async_copy(src0_hbm.at[pix.at[b, gcol]], rows[b],
                                      gsems[b]).wait()
                dss.append(
                    pltpu.async_copy(
                        rows[b], acc.at[pix.at[b, 1 - gcol]], ssems[b],
                        add=True,
                    )
                )
            if with_counts:
                @pl.when(c == 0)
                def _():
                    for b in range(NB):
                        pltpu.sync_copy(ones, cnt.at[pix.at[b, 0]], add=True)

                @pl.when(c == 1)
                def _():
                    for b in range(NB):
                        pltpu.sync_copy(ones, cnt.at[pix.at[b, 1]], add=True)
            for d in dss:
                d.wait()

        plsc.subcore_barrier()

        sl = pl.ds(s * ZSTR, ZSTR)

        @pl.when(c == 0)
        def _():
            pltpu.sync_copy(acc.at[sl], half0.at[sl])
            if with_counts:
                pltpu.sync_copy(cnt.at[sl], dcnt_out.at[sl])

        @pl.when(c == 1)
        def _():
            pltpu.sync_copy(acc.at[sl], half1.at[sl])
            if with_counts:
                pltpu.sync_copy(cnt.at[sl], bcnt_out.at[sl])

    return pl.kernel(
        body,
        out_type=tuple(outs) if with_counts else tuple(outs),
        mesh=_mesh,
        scratch_types=scratch,
        compiler_params=_params,
    )


def _make_combine_kernel():
    cstr = 512
    nws = NPAD // cstr
    outs = (
        jax.ShapeDtypeStruct((NPAD, DH), jnp.float32),
        jax.ShapeDtypeStruct((NPAD, DH), jnp.float32),
        jax.ShapeDtypeStruct((NPAD,), jnp.float32),
    )
    scratch = [
        pltpu.VMEM((CH, DH), jnp.float32),
        pltpu.VMEM((CH, DH), jnp.float32),
        pltpu.VMEM((cstr,), jnp.float32),
        pltpu.VMEM((cstr,), jnp.float32),
    ]

    def body(eh0, eh1, dcnt, bcnt, ef0, ef1, dinv, p0, p1, binv, dbuf):
        c, s = _worker()
        w = s * NC + c
        e0 = w * cstr

        @pl.when(w < nws)
        def _():
            pltpu.sync_copy(bcnt.at[pl.ds(e0, cstr)], binv)
            pltpu.sync_copy(dcnt.at[pl.ds(e0, cstr)], dbuf)

            @pl.loop(0, cstr // 16)
            def _(i):
                sl = pl.ds(i * 16, 16)
                b = binv[sl]
                binv[sl] = jnp.where(b > 0.0, 1.0 / b, 0.0)
                d = dbuf[sl]
                dbuf[sl] = jnp.where(d > 0.0, 1.0 / d, 0.0)

            pltpu.sync_copy(dbuf, dinv.at[pl.ds(e0, cstr)])
            zi = jnp.zeros((16,), jnp.int32)

            @pl.loop(0, cstr // CH)
            def _(q):
                r0 = e0 + q * CH
                pltpu.sync_copy(eh0.at[pl.ds(r0, CH)], p0)
                pltpu.sync_copy(eh1.at[pl.ds(r0, CH)], p1)

                @pl.loop(0, CH)
                def _(i):
                    bs = plsc.load_gather(binv, [zi + (q * CH + i)])
                    for k in range(DH // 16):
                        sl = pl.ds(k * 16, 16)
                        p0[i, sl] = p0[i, sl] * bs
                        p1[i, sl] = p1[i, sl] * bs

                pltpu.sync_copy(p0, ef0.at[pl.ds(r0, CH)])
                pltpu.sync_copy(p1, ef1.at[pl.ds(r0, CH)])

    return pl.kernel(body, out_type=outs, mesh=_mesh, scratch_types=scratch,
                     compiler_params=_params)


def _mm_body(x_ref, w_ref, o0_ref, o1_ref):
    y = jnp.dot(x_ref[...], w_ref[...], preferred_element_type=jnp.float32)
    o0_ref[...] = y[:, :DH]
    o1_ref[...] = y[:, DH:]


def _out_body(q0_ref, q1_ref, dinv_ref, b_ref, o_ref):
    r = jnp.concatenate([q0_ref[...], q1_ref[...]], axis=1)
    r = r * dinv_ref[...] + b_ref[...]
    m = jnp.max(r, axis=1, keepdims=True)
    e = jnp.exp(r - m)
    o_ref[...] = e / jnp.sum(e, axis=1, keepdims=True)


def kernel(X, edge_index, W, bias):
    n, d_in = X.shape
    d_out = W.shape[1]
    e = edge_index.shape[1]
    unit = NS * NB * CH
    epad = ((e + unit - 1) // unit) * unit
    nchunk = epad // (NS * CH)

    Xp = jnp.concatenate(
        [X, jnp.zeros((NPAD - n, d_in), jnp.float32)], axis=0
    )
    pad_ids = n + (jnp.arange(epad - e, dtype=jnp.int32) % (NPAD - n))
    ei = jnp.concatenate(
        [edge_index, jnp.stack([pad_ids, pad_ids])], axis=1
    )
    pidx = jnp.stack(
        [ei[0].reshape(NS, nchunk, CH), ei[1].reshape(NS, nchunk, CH)],
        axis=2,
    )

    rb = 512
    x0, x1 = pl.pallas_call(
        _mm_body,
        grid=(NPAD // rb,),
        in_specs=[
            pl.BlockSpec((rb, d_in), lambda i: (i, 0)),
            pl.BlockSpec((d_in, d_out), lambda i: (0, 0)),
        ],
        out_specs=[
            pl.BlockSpec((rb, DH), lambda i: (i, 0)),
            pl.BlockSpec((rb, DH), lambda i: (i, 0)),
        ],
        out_shape=[
            jax.ShapeDtypeStruct((NPAD, DH), jnp.float32),
            jax.ShapeDtypeStruct((NPAD, DH), jnp.float32),
        ],
    )(Xp, W)

    eh0, eh1, dcnt, bcnt = _make_scatter_kernel(nchunk, 0, True)(x0, x1, pidx)

    ef0, ef1, dinv = _make_combine_kernel()(eh0, eh1, dcnt, bcnt)

    oh0, oh1 = _make_scatter_kernel(nchunk, 1, False)(ef0, ef1, pidx)

    ob = 400
    dinv_col = dinv.reshape(NPAD, 1)
    bias2 = bias.reshape(1, d_out)
    out = pl.pallas_call(
        _out_body,
        grid=(n // ob,),
        in_specs=[
            pl.BlockSpec((ob, DH), lambda i: (i, 0)),
            pl.BlockSpec((ob, DH), lambda i: (i, 0)),
            pl.BlockSpec((ob, 1), lambda i: (i, 0)),
            pl.BlockSpec((1, d_out), lambda i: (0, 0)),
        ],
        out_specs=pl.BlockSpec((ob, d_out), lambda i: (i, 0)),
        out_shape=jax.ShapeDtypeStruct((n, d_out), jnp.float32),
    )(oh0, oh1, dinv_col, bias2)
    return out


# speedup vs baseline: 1.1119x; 1.1119x over previous
import jax
import jax.numpy as jnp
from jax import lax
from jax.experimental import pallas as pl
from jax.experimental.pallas import tpu as pltpu
from jax.experimental.pallas import tpu_sc as plsc

D = 128
DH = D // 2
NC, NS = 2, 16
NPAD = 10240
ZSTR = NPAD // NS
CH = 128
NB = 4

_mesh = plsc.VectorSubcoreMesh(
    core_axis_name="c", subcore_axis_name="s", num_cores=NC, num_subcores=NS
)
_params = pltpu.CompilerParams(
    needs_layout_passes=False, use_tc_tiling_on_sc=False
)


def _worker():
    return lax.axis_index("c"), lax.axis_index("s")


def _fill_1d(ref, n, val):
    v16 = jnp.full((16,), val, jnp.float32)

    @pl.loop(0, n // 16)
    def _(i):
        ref[pl.ds(i * 16, 16)] = v16


def _make_scatter_kernel(nchunk, gcol, with_counts):
    outs = [
        jax.ShapeDtypeStruct((NPAD, DH), jnp.float32),
        jax.ShapeDtypeStruct((NPAD, DH), jnp.float32),
    ]
    if with_counts:
        outs += [
            jax.ShapeDtypeStruct((NPAD,), jnp.float32),
            jax.ShapeDtypeStruct((NPAD,), jnp.float32),
        ]
    scratch = (
        [pltpu.VMEM_SHARED((NPAD, DH), jnp.float32)]
        + [pltpu.VMEM_SHARED((NPAD, DH), jnp.float32)]
        + [pltpu.VMEM((NB, 2, CH), jnp.int32)]
        + [pltpu.VMEM((CH, DH), jnp.float32) for _ in range(NB)]
        + [pltpu.SemaphoreType.DMA]
        + [pltpu.SemaphoreType.DMA for _ in range(NB)]
        + [pltpu.SemaphoreType.DMA for _ in range(NB)]
    )
    if with_counts:
        scratch += [
            pltpu.VMEM_SHARED((NPAD,), jnp.float32),
            pltpu.VMEM((CH,), jnp.float32),
            pltpu.VMEM((ZSTR,), jnp.float32),
        ]

    def body(src0_hbm, src1_hbm, pidx_hbm, *rest):
        if with_counts:
            (half0, half1, dcnt_out, bcnt_out, acc, xsp, pix, *rest2) = rest
            rows = rest2[:NB]
            isem = rest2[NB]
            gsems = rest2[NB + 1:2 * NB + 1]
            ssems = rest2[2 * NB + 1:3 * NB + 1]
            cnt, ones, zcnt = rest2[3 * NB + 1:]
        else:
            (half0, half1, acc, xsp, pix, *rest2) = rest
            rows = rest2[:NB]
            isem = rest2[NB]
            gsems = rest2[NB + 1:2 * NB + 1]
            ssems = rest2[2 * NB + 1:3 * NB + 1]
        c, s = _worker()
        z16 = jnp.zeros((16,), jnp.float32)

        @pl.loop(0, CH * (DH // 16))
        def _(t):
            rows[0][t // (DH // 16), pl.ds((t % (DH // 16)) * 16, 16)] = z16

        @pl.loop(0, ZSTR // CH)
        def _(q):
            pltpu.sync_copy(rows[0], acc.at[pl.ds(s * ZSTR + q * CH, CH)])

        if with_counts:
            _fill_1d(ones, CH, 1.0)
            _fill_1d(zcnt, ZSTR, 0.0)
            pltpu.sync_copy(zcnt, cnt.at[pl.ds(s * ZSTR, ZSTR)])

        stl = pl.ds(s * ZSTR, ZSTR)

        @pl.when(c == 0)
        def _():
            pltpu.sync_copy(src0_hbm.at[stl], xsp.at[stl])

        @pl.when(c == 1)
        def _():
            pltpu.sync_copy(src1_hbm.at[stl], xsp.at[stl])

        plsc.subcore_barrier()

        @pl.loop(0, nchunk // NB)
        def _(gq):
            j0 = gq * NB
            pltpu.async_copy(pidx_hbm.at[s, pl.ds(j0, NB)], pix, isem).wait()
            dgs = []
            for b in range(NB):
                dgs.append(
                    pltpu.async_copy(xsp.at[pix.at[b, gcol]], rows[b],
                                     gsems[b])
                )
            dss = []
            for b in range(NB):
                dgs[b].wait()
                dss.append(
                    pltpu.async_copy(
                        rows[b], acc.at[pix.at[b, 1 - gcol]], ssems[b],
                        add=True,
                    )
                )
            if with_counts:
                @pl.when(c == 0)
                def _():
                    for b in range(NB):
                        pltpu.sync_copy(ones, cnt.at[pix.at[b, 0]], add=True)

                @pl.when(c == 1)
                def _():
                    for b in range(NB):
                        pltpu.sync_copy(ones, cnt.at[pix.at[b, 1]], add=True)
            for d in dss:
                d.wait()

        plsc.subcore_barrier()

        sl = pl.ds(s * ZSTR, ZSTR)

        @pl.when(c == 0)
        def _():
            pltpu.sync_copy(acc.at[sl], half0.at[sl])
            if with_counts:
                pltpu.sync_copy(cnt.at[sl], dcnt_out.at[sl])

        @pl.when(c == 1)
        def _():
            pltpu.sync_copy(acc.at[sl], half1.at[sl])
            if with_counts:
                pltpu.sync_copy(cnt.at[sl], bcnt_out.at[sl])

    return pl.kernel(
        body,
        out_type=tuple(outs) if with_counts else tuple(outs),
        mesh=_mesh,
        scratch_types=scratch,
        compiler_params=_params,
    )


def _make_combine_kernel():
    cstr = 512
    nws = NPAD // cstr
    outs = (
        jax.ShapeDtypeStruct((NPAD, DH), jnp.float32),
        jax.ShapeDtypeStruct((NPAD, DH), jnp.float32),
        jax.ShapeDtypeStruct((NPAD,), jnp.float32),
    )
    scratch = [
        pltpu.VMEM((CH, DH), jnp.float32),
        pltpu.VMEM((CH, DH), jnp.float32),
        pltpu.VMEM((cstr,), jnp.float32),
        pltpu.VMEM((cstr,), jnp.float32),
    ]

    def body(eh0, eh1, dcnt, bcnt, ef0, ef1, dinv, p0, p1, binv, dbuf):
        c, s = _worker()
        w = s * NC + c
        e0 = w * cstr

        @pl.when(w < nws)
        def _():
            pltpu.sync_copy(bcnt.at[pl.ds(e0, cstr)], binv)
            pltpu.sync_copy(dcnt.at[pl.ds(e0, cstr)], dbuf)

            @pl.loop(0, cstr // 16)
            def _(i):
                sl = pl.ds(i * 16, 16)
                b = binv[sl]
                binv[sl] = jnp.where(b > 0.0, 1.0 / b, 0.0)
                d = dbuf[sl]
                dbuf[sl] = jnp.where(d > 0.0, 1.0 / d, 0.0)

            pltpu.sync_copy(dbuf, dinv.at[pl.ds(e0, cstr)])
            zi = jnp.zeros((16,), jnp.int32)

            @pl.loop(0, cstr // CH)
            def _(q):
                r0 = e0 + q * CH
                pltpu.sync_copy(eh0.at[pl.ds(r0, CH)], p0)
                pltpu.sync_copy(eh1.at[pl.ds(r0, CH)], p1)

                @pl.loop(0, CH)
                def _(i):
                    bs = plsc.load_gather(binv, [zi + (q * CH + i)])
                    for k in range(DH // 16):
                        sl = pl.ds(k * 16, 16)
                        p0[i, sl] = p0[i, sl] * bs
                        p1[i, sl] = p1[i, sl] * bs

                pltpu.sync_copy(p0, ef0.at[pl.ds(r0, CH)])
                pltpu.sync_copy(p1, ef1.at[pl.ds(r0, CH)])

    return pl.kernel(body, out_type=outs, mesh=_mesh, scratch_types=scratch,
                     compiler_params=_params)


def _mm_body(x_ref, w_ref, o0_ref, o1_ref):
    y = jnp.dot(x_ref[...], w_ref[...], preferred_element_type=jnp.float32)
    o0_ref[...] = y[:, :DH]
    o1_ref[...] = y[:, DH:]


def _out_body(q0_ref, q1_ref, dinv_ref, b_ref, o_ref):
    r = jnp.concatenate([q0_ref[...], q1_ref[...]], axis=1)
    r = r * dinv_ref[...] + b_ref[...]
    m = jnp.max(r, axis=1, keepdims=True)
    e = jnp.exp(r - m)
    o_ref[...] = e / jnp.sum(e, axis=1, keepdims=True)


def kernel(X, edge_index, W, bias):
    n, d_in = X.shape
    d_out = W.shape[1]
    e = edge_index.shape[1]
    unit = NS * NB * CH
    epad = ((e + unit - 1) // unit) * unit
    nchunk = epad // (NS * CH)

    Xp = jnp.concatenate(
        [X, jnp.zeros((NPAD - n, d_in), jnp.float32)], axis=0
    )
    pad_ids = n + (jnp.arange(epad - e, dtype=jnp.int32) % (NPAD - n))
    ei = jnp.concatenate(
        [edge_index, jnp.stack([pad_ids, pad_ids])], axis=1
    )
    pidx = jnp.stack(
        [ei[0].reshape(NS, nchunk, CH), ei[1].reshape(NS, nchunk, CH)],
        axis=2,
    )

    rb = 512
    x0, x1 = pl.pallas_call(
        _mm_body,
        grid=(NPAD // rb,),
        in_specs=[
            pl.BlockSpec((rb, d_in), lambda i: (i, 0)),
            pl.BlockSpec((d_in, d_out), lambda i: (0, 0)),
        ],
        out_specs=[
            pl.BlockSpec((rb, DH), lambda i: (i, 0)),
            pl.BlockSpec((rb, DH), lambda i: (i, 0)),
        ],
        out_shape=[
            jax.ShapeDtypeStruct((NPAD, DH), jnp.float32),
            jax.ShapeDtypeStruct((NPAD, DH), jnp.float32),
        ],
    )(Xp, W)

    eh0, eh1, dcnt, bcnt = _make_scatter_kernel(nchunk, 0, True)(x0, x1, pidx)

    ef0, ef1, dinv = _make_combine_kernel()(eh0, eh1, dcnt, bcnt)

    oh0, oh1 = _make_scatter_kernel(nchunk, 1, False)(ef0, ef1, pidx)

    ob = 400
    dinv_col = dinv.reshape(NPAD, 1)
    bias2 = bias.reshape(1, d_out)
    out = pl.pallas_call(
        _out_body,
        grid=(n // ob,),
        in_specs=[
            pl.BlockSpec((ob, DH), lambda i: (i, 0)),
            pl.BlockSpec((ob, DH), lambda i: (i, 0)),
            pl.BlockSpec((ob, 1), lambda i: (i, 0)),
            pl.BlockSpec((1, d_out), lambda i: (0, 0)),
        ],
        out_specs=pl.BlockSpec((ob, d_out), lambda i: (i, 0)),
        out_shape=jax.ShapeDtypeStruct((n, d_out), jnp.float32),
    )(oh0, oh1, dinv_col, bias2)
    return out


# speedup vs baseline: 1.1142x; 1.0020x over previous
import jax
import jax.numpy as jnp
from jax import lax
from jax.experimental import pallas as pl
from jax.experimental.pallas import tpu as pltpu
from jax.experimental.pallas import tpu_sc as plsc

D = 128
DH = D // 2
NC, NS = 2, 16
NPAD = 10240
ZSTR = NPAD // NS
CH = 128
NB = 4

_mesh = plsc.VectorSubcoreMesh(
    core_axis_name="c", subcore_axis_name="s", num_cores=NC, num_subcores=NS
)
_params = pltpu.CompilerParams(
    needs_layout_passes=False, use_tc_tiling_on_sc=False
)


def _worker():
    return lax.axis_index("c"), lax.axis_index("s")


def _fill_1d(ref, n, val):
    v16 = jnp.full((16,), val, jnp.float32)

    @pl.loop(0, n // 16)
    def _(i):
        ref[pl.ds(i * 16, 16)] = v16


def _make_scatter_kernel(nchunk, gcol, with_counts):
    outs = [
        jax.ShapeDtypeStruct((NPAD, DH), jnp.float32),
        jax.ShapeDtypeStruct((NPAD, DH), jnp.float32),
    ]
    if with_counts:
        outs += [
            jax.ShapeDtypeStruct((NPAD,), jnp.float32),
        ]
    scratch = (
        [pltpu.VMEM_SHARED((NPAD, DH), jnp.float32)]
        + [pltpu.VMEM_SHARED((NPAD, DH), jnp.float32)]
        + [pltpu.VMEM((NB, 2, CH), jnp.int32)]
        + [pltpu.VMEM((CH, DH), jnp.float32) for _ in range(NB)]
        + [pltpu.SemaphoreType.DMA]
        + [pltpu.SemaphoreType.DMA for _ in range(NB)]
        + [pltpu.SemaphoreType.DMA for _ in range(NB)]
    )
    if with_counts:
        scratch += [
            pltpu.VMEM_SHARED((NPAD,), jnp.float32),
            pltpu.VMEM_SHARED((NPAD,), jnp.float32),
            pltpu.VMEM((CH,), jnp.float32),
            pltpu.VMEM((ZSTR,), jnp.float32),
        ]

    def body(src0_hbm, src1_hbm, pidx_hbm, *rest):
        if with_counts:
            (half0, half1, dinv_out, acc, xsp, pix, *rest2) = rest
            rows = rest2[:NB]
            isem = rest2[NB]
            gsems = rest2[NB + 1:2 * NB + 1]
            ssems = rest2[2 * NB + 1:3 * NB + 1]
            cnt_d, cnt_b, ones, zcnt = rest2[3 * NB + 1:]
        else:
            (half0, half1, acc, xsp, pix, *rest2) = rest
            rows = rest2[:NB]
            isem = rest2[NB]
            gsems = rest2[NB + 1:2 * NB + 1]
            ssems = rest2[2 * NB + 1:3 * NB + 1]
        c, s = _worker()
        z16 = jnp.zeros((16,), jnp.float32)

        @pl.loop(0, CH * (DH // 16))
        def _(t):
            rows[0][t // (DH // 16), pl.ds((t % (DH // 16)) * 16, 16)] = z16

        @pl.loop(0, ZSTR // CH)
        def _(q):
            pltpu.sync_copy(rows[0], acc.at[pl.ds(s * ZSTR + q * CH, CH)])

        if with_counts:
            _fill_1d(ones, CH, 1.0)
            _fill_1d(zcnt, ZSTR, 0.0)
            pltpu.sync_copy(zcnt, cnt_d.at[pl.ds(s * ZSTR, ZSTR)])
            pltpu.sync_copy(zcnt, cnt_b.at[pl.ds(s * ZSTR, ZSTR)])

        stl = pl.ds(s * ZSTR, ZSTR)

        @pl.when(c == 0)
        def _():
            pltpu.sync_copy(src0_hbm.at[stl], xsp.at[stl])

        @pl.when(c == 1)
        def _():
            pltpu.sync_copy(src1_hbm.at[stl], xsp.at[stl])

        plsc.subcore_barrier()

        @pl.loop(0, nchunk // NB)
        def _(gq):
            j0 = gq * NB
            pltpu.async_copy(pidx_hbm.at[s, pl.ds(j0, NB)], pix, isem).wait()
            dgs = []
            for b in range(NB):
                dgs.append(
                    pltpu.async_copy(xsp.at[pix.at[b, gcol]], rows[b],
                                     gsems[b])
                )
            dss = []
            for b in range(NB):
                dgs[b].wait()
                dss.append(
                    pltpu.async_copy(
                        rows[b], acc.at[pix.at[b, 1 - gcol]], ssems[b],
                        add=True,
                    )
                )
            if with_counts:
                for b in range(NB):
                    pltpu.sync_copy(ones, cnt_d.at[pix.at[b, 0]], add=True)
                    pltpu.sync_copy(ones, cnt_b.at[pix.at[b, 1]], add=True)
            for d in dss:
                d.wait()

        plsc.subcore_barrier()

        sl = pl.ds(s * ZSTR, ZSTR)
        if with_counts:

            def _recip_inplace():
                @pl.loop(0, ZSTR // 16)
                def _(i):
                    slv = pl.ds(i * 16, 16)
                    v = zcnt[slv]
                    zcnt[slv] = jnp.where(v > 0.0, 1.0 / v, 0.0)

            @pl.when(c == 0)
            def _():
                pltpu.sync_copy(cnt_d.at[sl], zcnt)
                _recip_inplace()
                pltpu.sync_copy(zcnt, dinv_out.at[sl])

            pltpu.sync_copy(cnt_b.at[sl], zcnt)
            _recip_inplace()
            zi = jnp.zeros((16,), jnp.int32)

            @pl.loop(0, ZSTR // CH)
            def _(q):
                rsl = pl.ds(s * ZSTR + q * CH, CH)
                pltpu.sync_copy(acc.at[rsl], rows[0])

                @pl.loop(0, CH)
                def _(i):
                    bs = plsc.load_gather(zcnt, [zi + (q * CH + i)])
                    for k in range(DH // 16):
                        ksl = pl.ds(k * 16, 16)
                        rows[0][i, ksl] = rows[0][i, ksl] * bs

                @pl.when(c == 0)
                def _():
                    pltpu.sync_copy(rows[0], half0.at[rsl])

                @pl.when(c == 1)
                def _():
                    pltpu.sync_copy(rows[0], half1.at[rsl])
        else:

            @pl.when(c == 0)
            def _():
                pltpu.sync_copy(acc.at[sl], half0.at[sl])

            @pl.when(c == 1)
            def _():
                pltpu.sync_copy(acc.at[sl], half1.at[sl])

    return pl.kernel(
        body,
        out_type=tuple(outs) if with_counts else tuple(outs),
        mesh=_mesh,
        scratch_types=scratch,
        compiler_params=_params,
    )


def _mm_body(x_ref, w_ref, o0_ref, o1_ref):
    y = jnp.dot(x_ref[...], w_ref[...], preferred_element_type=jnp.float32)
    o0_ref[...] = y[:, :DH]
    o1_ref[...] = y[:, DH:]


def _out_body(q0_ref, q1_ref, dinv_ref, b_ref, o_ref):
    r = jnp.concatenate([q0_ref[...], q1_ref[...]], axis=1)
    r = r * dinv_ref[...] + b_ref[...]
    m = jnp.max(r, axis=1, keepdims=True)
    e = jnp.exp(r - m)
    o_ref[...] = e / jnp.sum(e, axis=1, keepdims=True)


def kernel(X, edge_index, W, bias):
    n, d_in = X.shape
    d_out = W.shape[1]
    e = edge_index.shape[1]
    unit = NS * NB * CH
    epad = ((e + unit - 1) // unit) * unit
    nchunk = epad // (NS * CH)

    Xp = jnp.concatenate(
        [X, jnp.zeros((NPAD - n, d_in), jnp.float32)], axis=0
    )
    pad_ids = n + (jnp.arange(epad - e, dtype=jnp.int32) % (NPAD - n))
    ei = jnp.concatenate(
        [edge_index, jnp.stack([pad_ids, pad_ids])], axis=1
    )
    pidx = jnp.stack(
        [ei[0].reshape(NS, nchunk, CH), ei[1].reshape(NS, nchunk, CH)],
        axis=2,
    )

    rb = 512
    x0, x1 = pl.pallas_call(
        _mm_body,
        grid=(NPAD // rb,),
        in_specs=[
            pl.BlockSpec((rb, d_in), lambda i: (i, 0)),
            pl.BlockSpec((d_in, d_out), lambda i: (0, 0)),
        ],
        out_specs=[
            pl.BlockSpec((rb, DH), lambda i: (i, 0)),
            pl.BlockSpec((rb, DH), lambda i: (i, 0)),
        ],
        out_shape=[
            jax.ShapeDtypeStruct((NPAD, DH), jnp.float32),
            jax.ShapeDtypeStruct((NPAD, DH), jnp.float32),
        ],
    )(Xp, W)

    ef0, ef1, dinv = _make_scatter_kernel(nchunk, 0, True)(x0, x1, pidx)

    oh0, oh1 = _make_scatter_kernel(nchunk, 1, False)(ef0, ef1, pidx)

    ob = 400
    dinv_col = dinv.reshape(NPAD, 1)
    bias2 = bias.reshape(1, d_out)
    out = pl.pallas_call(
        _out_body,
        grid=(n // ob,),
        in_specs=[
            pl.BlockSpec((ob, DH), lambda i: (i, 0)),
            pl.BlockSpec((ob, DH), lambda i: (i, 0)),
            pl.BlockSpec((ob, 1), lambda i: (i, 0)),
            pl.BlockSpec((1, d_out), lambda i: (0, 0)),
        ],
        out_specs=pl.BlockSpec((ob, d_out), lambda i: (i, 0)),
        out_shape=jax.ShapeDtypeStruct((n, d_out), jnp.float32),
    )(oh0, oh1, dinv_col, bias2)
    return out


# speedup vs baseline: 1.2185x; 1.0936x over previous
import jax
import jax.numpy as jnp
from jax import lax
from jax.experimental import pallas as pl
from jax.experimental.pallas import tpu as pltpu
from jax.experimental.pallas import tpu_sc as plsc

D = 128
DH = D // 2
NC, NS = 2, 16
NPAD = 10240
ZSTR = NPAD // NS
CH = 128
NB = 4

_mesh = plsc.VectorSubcoreMesh(
    core_axis_name="c", subcore_axis_name="s", num_cores=NC, num_subcores=NS
)
_params = pltpu.CompilerParams(
    needs_layout_passes=False, use_tc_tiling_on_sc=False
)


def _worker():
    return lax.axis_index("c"), lax.axis_index("s")


def _fill_1d(ref, n, val):
    v16 = jnp.full((16,), val, jnp.float32)

    @pl.loop(0, n // 16)
    def _(i):
        ref[pl.ds(i * 16, 16)] = v16


def _make_scatter_kernel(nchunk, gcol, with_counts):
    outs = [
        jax.ShapeDtypeStruct((NPAD, DH), jnp.float32),
        jax.ShapeDtypeStruct((NPAD, DH), jnp.float32),
    ]
    if with_counts:
        outs += [
            jax.ShapeDtypeStruct((NPAD,), jnp.float32),
        ]
    scratch = (
        [pltpu.VMEM_SHARED((NPAD, DH), jnp.float32)]
        + [pltpu.VMEM_SHARED((NPAD, DH), jnp.float32)]
        + [pltpu.VMEM((NB, 2, CH), jnp.int32)]
        + [pltpu.VMEM((CH, DH), jnp.float32) for _ in range(NB)]
        + [pltpu.SemaphoreType.DMA]
        + [pltpu.SemaphoreType.DMA for _ in range(NB)]
        + [pltpu.SemaphoreType.DMA for _ in range(NB)]
    )
    if with_counts:
        scratch += [
            pltpu.VMEM_SHARED((NPAD,), jnp.float32),
            pltpu.VMEM_SHARED((NPAD,), jnp.float32),
            pltpu.VMEM((CH,), jnp.float32),
            pltpu.VMEM((ZSTR,), jnp.float32),
            pltpu.SemaphoreType.DMA,
            pltpu.SemaphoreType.DMA,
        ]

    def body(src0_hbm, src1_hbm, pidx_hbm, *rest):
        if with_counts:
            (half0, half1, dinv_out, acc, xsp, pix, *rest2) = rest
            rows = rest2[:NB]
            isem = rest2[NB]
            gsems = rest2[NB + 1:2 * NB + 1]
            ssems = rest2[2 * NB + 1:3 * NB + 1]
            cnt_d, cnt_b, ones, zcnt, dsem, bsem = rest2[3 * NB + 1:]
        else:
            (half0, half1, acc, xsp, pix, *rest2) = rest
            rows = rest2[:NB]
            isem = rest2[NB]
            gsems = rest2[NB + 1:2 * NB + 1]
            ssems = rest2[2 * NB + 1:3 * NB + 1]
        c, s = _worker()
        z16 = jnp.zeros((16,), jnp.float32)

        @pl.loop(0, CH * (DH // 16))
        def _(t):
            rows[0][t // (DH // 16), pl.ds((t % (DH // 16)) * 16, 16)] = z16

        @pl.loop(0, ZSTR // CH)
        def _(q):
            pltpu.sync_copy(rows[0], acc.at[pl.ds(s * ZSTR + q * CH, CH)])

        if with_counts:
            _fill_1d(ones, CH, 1.0)
            _fill_1d(zcnt, ZSTR, 0.0)
            pltpu.sync_copy(zcnt, cnt_d.at[pl.ds(s * ZSTR, ZSTR)])
            pltpu.sync_copy(zcnt, cnt_b.at[pl.ds(s * ZSTR, ZSTR)])

        stl = pl.ds(s * ZSTR, ZSTR)

        @pl.when(c == 0)
        def _():
            pltpu.sync_copy(src0_hbm.at[stl], xsp.at[stl])

        @pl.when(c == 1)
        def _():
            pltpu.sync_copy(src1_hbm.at[stl], xsp.at[stl])

        plsc.subcore_barrier()

        @pl.loop(0, nchunk // NB)
        def _(gq):
            j0 = gq * NB
            pltpu.async_copy(pidx_hbm.at[s, pl.ds(j0, NB)], pix, isem).wait()
            dgs = []
            for b in range(NB):
                if b % 2 == 0:
                    dgs.append(
                        pltpu.async_copy(xsp.at[pix.at[b, gcol]], rows[b],
                                         gsems[b])
                    )
                else:
                    @pl.when(c == 0)
                    def _(b=b):
                        pltpu.async_copy(src0_hbm.at[pix.at[b, gcol]],
                                         rows[b], gsems[b])

                    @pl.when(c == 1)
                    def _(b=b):
                        pltpu.async_copy(src1_hbm.at[pix.at[b, gcol]],
                                         rows[b], gsems[b])
                    dgs.append(
                        pltpu.make_async_copy(src0_hbm.at[pix.at[b, gcol]],
                                              rows[b], gsems[b])
                    )
            dcs = []
            if with_counts:
                for b in range(NB):
                    dcs.append(
                        pltpu.async_copy(ones, cnt_d.at[pix.at[b, 0]],
                                         dsem, add=True)
                    )
                    dcs.append(
                        pltpu.async_copy(ones, cnt_b.at[pix.at[b, 1]],
                                         bsem, add=True)
                    )
            dss = []
            for b in range(NB):
                dgs[b].wait()
                dss.append(
                    pltpu.async_copy(
                        rows[b], acc.at[pix.at[b, 1 - gcol]], ssems[b],
                        add=True,
                    )
                )
            for d in dss:
                d.wait()
            for d in dcs:
                d.wait()

        plsc.subcore_barrier()

        sl = pl.ds(s * ZSTR, ZSTR)
        if with_counts:

            def _recip_inplace():
                @pl.loop(0, ZSTR // 16)
                def _(i):
                    slv = pl.ds(i * 16, 16)
                    v = zcnt[slv]
                    zcnt[slv] = jnp.where(v > 0.0, 1.0 / v, 0.0)

            @pl.when(c == 0)
            def _():
                pltpu.sync_copy(cnt_d.at[sl], zcnt)
                _recip_inplace()
                pltpu.sync_copy(zcnt, dinv_out.at[sl])

            pltpu.sync_copy(cnt_b.at[sl], zcnt)
            _recip_inplace()
            zi = jnp.zeros((16,), jnp.int32)

            @pl.loop(0, ZSTR // CH)
            def _(q):
                rsl = pl.ds(s * ZSTR + q * CH, CH)
                pltpu.sync_copy(acc.at[rsl], rows[0])

                @pl.loop(0, CH)
                def _(i):
                    bs = plsc.load_gather(zcnt, [zi + (q * CH + i)])
                    for k in range(DH // 16):
                        ksl = pl.ds(k * 16, 16)
                        rows[0][i, ksl] = rows[0][i, ksl] * bs

                @pl.when(c == 0)
                def _():
                    pltpu.sync_copy(rows[0], half0.at[rsl])

                @pl.when(c == 1)
                def _():
                    pltpu.sync_copy(rows[0], half1.at[rsl])
        else:

            @pl.when(c == 0)
            def _():
                pltpu.sync_copy(acc.at[sl], half0.at[sl])

            @pl.when(c == 1)
            def _():
                pltpu.sync_copy(acc.at[sl], half1.at[sl])

    return pl.kernel(
        body,
        out_type=tuple(outs) if with_counts else tuple(outs),
        mesh=_mesh,
        scratch_types=scratch,
        compiler_params=_params,
    )


def _mm_body(x_ref, w_ref, o0_ref, o1_ref):
    y = jnp.dot(x_ref[...], w_ref[...], preferred_element_type=jnp.float32)
    o0_ref[...] = y[:, :DH]
    o1_ref[...] = y[:, DH:]


def _out_body(q0_ref, q1_ref, dinv_ref, b_ref, o_ref):
    r = jnp.concatenate([q0_ref[...], q1_ref[...]], axis=1)
    r = r * dinv_ref[...] + b_ref[...]
    m = jnp.max(r, axis=1, keepdims=True)
    e = jnp.exp(r - m)
    o_ref[...] = e / jnp.sum(e, axis=1, keepdims=True)


def kernel(X, edge_index, W, bias):
    n, d_in = X.shape
    d_out = W.shape[1]
    e = edge_index.shape[1]
    unit = NS * NB * CH
    epad = ((e + unit - 1) // unit) * unit
    nchunk = epad // (NS * CH)

    Xp = jnp.concatenate(
        [X, jnp.zeros((NPAD - n, d_in), jnp.float32)], axis=0
    )
    pad_ids = n + (jnp.arange(epad - e, dtype=jnp.int32) % (NPAD - n))
    ei = jnp.concatenate(
        [edge_index, jnp.stack([pad_ids, pad_ids])], axis=1
    )
    pidx = jnp.stack(
        [ei[0].reshape(NS, nchunk, CH), ei[1].reshape(NS, nchunk, CH)],
        axis=2,
    )

    rb = 512
    x0, x1 = pl.pallas_call(
        _mm_body,
        grid=(NPAD // rb,),
        in_specs=[
            pl.BlockSpec((rb, d_in), lambda i: (i, 0)),
            pl.BlockSpec((d_in, d_out), lambda i: (0, 0)),
        ],
        out_specs=[
            pl.BlockSpec((rb, DH), lambda i: (i, 0)),
            pl.BlockSpec((rb, DH), lambda i: (i, 0)),
        ],
        out_shape=[
            jax.ShapeDtypeStruct((NPAD, DH), jnp.float32),
            jax.ShapeDtypeStruct((NPAD, DH), jnp.float32),
        ],
    )(Xp, W)

    ef0, ef1, dinv = _make_scatter_kernel(nchunk, 0, True)(x0, x1, pidx)

    oh0, oh1 = _make_scatter_kernel(nchunk, 1, False)(ef0, ef1, pidx)

    ob = 400
    dinv_col = dinv.reshape(NPAD, 1)
    bias2 = bias.reshape(1, d_out)
    out = pl.pallas_call(
        _out_body,
        grid=(n // ob,),
        in_specs=[
            pl.BlockSpec((ob, DH), lambda i: (i, 0)),
            pl.BlockSpec((ob, DH), lambda i: (i, 0)),
            pl.BlockSpec((ob, 1), lambda i: (i, 0)),
            pl.BlockSpec((1, d_out), lambda i: (0, 0)),
        ],
        out_specs=pl.BlockSpec((ob, d_out), lambda i: (i, 0)),
        out_shape=jax.ShapeDtypeStruct((n, d_out), jnp.float32),
    )(oh0, oh1, dinv_col, bias2)
    return out
